# Initial kernel scaffold; baseline (speedup 1.0000x reference)
import jax
import jax.numpy as jnp
from jax.experimental import pallas as pl


def kernel(points, links, density_data, sh_data):
    raise NotImplementedError("write your pallas kernel here")


# speedup vs baseline: 4.3313x; 4.3313x over previous
import functools

import jax
import jax.numpy as jnp
from jax import lax
from jax.experimental import pallas as pl
from jax.experimental.pallas import tpu as pltpu
from jax.experimental.pallas import tpu_sc as plsc

NC = 2
NS = 16
NW = NC * NS
LANES = 16

N = 256
GROUPS = N // LANES
NIDX = 8 * N
NSTREAM = 128
NDMA = NIDX // NSTREAM

DPAD = 32
DOUT = 28

CORNERS = [(dx, dy, dz) for dx in (0, 1) for dy in (0, 1) for dz in (0, 1)]


def _sc_body(RX, RY, RZ, CHUNKS, px_hbm, py_hbm, pz_hbm, links_hbm, data_hbm,
             out_hbm, px_v, py_v, pz_v, lidx_v, lvals_v, rows_v, w_v, outb_v,
             sem):
    wid = lax.axis_index("s") * NC + lax.axis_index("c")
    pw = CHUNKS * N

    def chunk_body(t, carry):
        base = wid * pw + t * N

        pltpu.sync_copy(px_hbm.at[pl.ds(base, N)], px_v)
        pltpu.sync_copy(py_hbm.at[pl.ds(base, N)], py_v)
        pltpu.sync_copy(pz_hbm.at[pl.ds(base, N)], pz_v)

        def grp_idx(i, c2):
            s = i * LANES
            fx = px_v[pl.ds(s, LANES)] * RX
            fy = py_v[pl.ds(s, LANES)] * RY
            fz = pz_v[pl.ds(s, LANES)] * RZ
            fx = jnp.minimum(jnp.maximum(fx, 0.0), RX)
            fy = jnp.minimum(jnp.maximum(fy, 0.0), RY)
            fz = jnp.minimum(jnp.maximum(fz, 0.0), RZ)
            lx = jnp.minimum(fx.astype(jnp.int32), int(RX) - 1)
            ly = jnp.minimum(fy.astype(jnp.int32), int(RY) - 1)
            lz = jnp.minimum(fz.astype(jnp.int32), int(RZ) - 1)
            wx = fx - lx.astype(jnp.float32)
            wy = fy - ly.astype(jnp.float32)
            wz = fz - lz.astype(jnp.float32)
            b3 = lx * ((int(RY) + 1) * (int(RZ) + 1)) \
                + ly * (int(RZ) + 1) + lz
            wx0 = 1.0 - wx
            wy0 = 1.0 - wy
            wz0 = 1.0 - wz
            for c, (dx, dy, dz) in enumerate(CORNERS):
                off = dx * ((int(RY) + 1) * (int(RZ) + 1)) \
                    + dy * (int(RZ) + 1) + dz
                lidx_v[pl.ds(c * N + s, LANES)] = b3 + off
                wprod = (wx if dx else wx0) * (wy if dy else wy0) \
                    * (wz if dz else wz0)
                w_v[pl.ds(c * N + s, LANES)] = wprod
            return c2

        lax.fori_loop(0, GROUPS, grp_idx, 0)

        cps = [
            pltpu.async_copy(
                links_hbm.at[lidx_v.at[pl.ds(r * NSTREAM, NSTREAM)]],
                lvals_v.at[pl.ds(r * NSTREAM, NSTREAM)], sem)
            for r in range(NDMA)
        ]
        for cp in cps:
            cp.wait()

        cps = [
            pltpu.async_copy(
                data_hbm.at[lvals_v.at[pl.ds(r * NSTREAM, NSTREAM)]],
                rows_v.at[pl.ds(r * NSTREAM, NSTREAM), :], sem)
            for r in range(NDMA)
        ]
        for cp in cps:
            cp.wait()

        def grp_acc(i, c2):
            s = i * LANES
            wvecs = [w_v[pl.ds(c * N + s, LANES)] for c in range(8)]
            for kk in range(LANES):
                k = s + kk
                acc0 = jnp.zeros((LANES,), jnp.float32)
                acc1 = jnp.zeros((LANES,), jnp.float32)
                for c in range(8):
                    ws = wvecs[c][kk]
                    acc0 = acc0 + ws * rows_v[c * N + k, pl.ds(0, LANES)]
                    acc1 = acc1 + ws * rows_v[c * N + k,
                                              pl.ds(DOUT - LANES, LANES)]
                outb_v[k, pl.ds(0, LANES)] = acc0
                outb_v[k, pl.ds(DOUT - LANES, LANES)] = acc1
            return c2

        lax.fori_loop(0, GROUPS, grp_acc, 0)

        pltpu.sync_copy(outb_v, out_hbm.at[pl.ds(base, N)])
        return carry

    lax.fori_loop(0, CHUNKS, chunk_body, 0)


def kernel(points, links, density_data, sh_data):
    B = points.shape[0]
    M = density_data.shape[0]
    rx, ry, rz = links.shape
    chunks = B // (NW * N)

    px = points[:, 0]
    py = points[:, 1]
    pz = points[:, 2]
    links_flat = links.reshape(-1)
    data = jnp.concatenate(
        [density_data, sh_data,
         jnp.zeros((M, DPAD - 1 - sh_data.shape[1]), jnp.float32)], axis=1)

    mesh = plsc.VectorSubcoreMesh(
        core_axis_name="c", subcore_axis_name="s",
        num_cores=NC, num_subcores=NS)
    body = functools.partial(
        _sc_body, float(rx - 1), float(ry - 1), float(rz - 1), chunks)
    f = pl.kernel(
        body,
        out_type=jax.ShapeDtypeStruct((B, DOUT), jnp.float32),
        mesh=mesh,
        compiler_params=pltpu.CompilerParams(use_tc_tiling_on_sc=False),
        scratch_types=[
            pltpu.VMEM((N,), jnp.float32),
            pltpu.VMEM((N,), jnp.float32),
            pltpu.VMEM((N,), jnp.float32),
            pltpu.VMEM((NIDX,), jnp.int32),
            pltpu.VMEM((NIDX,), jnp.int32),
            pltpu.VMEM((NIDX, DPAD), jnp.float32),
            pltpu.VMEM((NIDX,), jnp.float32),
            pltpu.VMEM((N, DOUT), jnp.float32),
            pltpu.SemaphoreType.DMA,
        ],
    )
    return f(px, py, pz, links_flat, data)
